# baseline (speedup 1.0000x reference)
import functools
import numpy as np
import jax
import jax.numpy as jnp
from jax import lax
from jax.experimental import pallas as pl
from jax.experimental.pallas import tpu as pltpu
from jax.experimental.pallas import tpu_sc as plsc

D_MODEL = 512
SEQ_LEN = 380
BATCH = 1024

NW = 32
BPW = BATCH // NW
CS = 32
NFULL = SEQ_LEN // CS
TAIL = SEQ_LEN - NFULL * CS
NCHUNK = NFULL + 1
SEQ_PAD = NCHUNK * CS
LANES = 16
KPR = D_MODEL // LANES
GPC = BPW // 4


def _pe_matrix_np(d_model, seq_len):
    a, b = np.meshgrid(np.arange(d_model), np.arange(seq_len))
    pe_mat = b / 10000 ** (2 * (a // 2) / d_model)
    pe_mat[:, 0::2] = np.sin(pe_mat[:, 0::2])
    pe_mat[:, 1::2] = np.cos(pe_mat[:, 1::2])
    return pe_mat.astype(np.float32)


@functools.partial(
    pl.kernel,
    out_type=jax.ShapeDtypeStruct((BATCH, SEQ_LEN, D_MODEL), jnp.float32),
    mesh=plsc.VectorSubcoreMesh(core_axis_name="c", subcore_axis_name="s"),
    scratch_types=[
        pltpu.VMEM((NCHUNK * GPC, 4 * CS), jnp.int32),
        pltpu.VMEM((CS, D_MODEL), jnp.float32),
        pltpu.VMEM((2 * CS, D_MODEL), jnp.float32),
        pltpu.VMEM((2 * CS, D_MODEL), jnp.float32),
        pltpu.VMEM((TAIL, D_MODEL), jnp.float32),
        pltpu.SemaphoreType.DMA,
        pltpu.SemaphoreType.DMA,
        pltpu.SemaphoreType.DMA,
        pltpu.SemaphoreType.DMA,
        pltpu.SemaphoreType.DMA,
        pltpu.SemaphoreType.DMA,
    ],
)
def _emb_pe_kernel(xc_hbm, table_hbm, pe_hbm, out_hbm,
                   idx_v, pe_v, rows0, rows1, tail_v,
                   gsem0, gsem1, ssem0, ssem1, ssem2, ssem3):
    rows = (rows0, rows1)
    gsem = (gsem0, gsem1)
    ssem = ((ssem0, ssem1), (ssem2, ssem3))

    wid = lax.axis_index("s") * 2 + lax.axis_index("c")
    pltpu.sync_copy(xc_hbm.at[wid], idx_v)

    NPAIR = BPW // 2

    def idx_list(cp, h, jj):
        return idx_v.at[cp * GPC + h, pl.ds(jj * 2 * CS, 2 * CS)]

    def issue_gather(cp, h, jj, b):
        pltpu.async_copy(table_hbm.at[idx_list(cp, h, jj)], rows[b], gsem[b])

    def wait_gather(cp, h, jj, b):
        pltpu.make_async_copy(
            table_hbm.at[idx_list(cp, h, jj)], rows[b], gsem[b]
        ).wait()

    def wait_scatter(b, half):
        pltpu.make_async_copy(
            rows[b].at[pl.ds(0, CS), :],
            out_hbm.at[0, pl.ds(0, CS), :],
            ssem[b][half],
        ).wait()

    def chunk_body(cp, carry):
        base = pl.multiple_of(cp * CS, CS)
        pltpu.sync_copy(pe_hbm.at[pl.ds(base, CS), :], pe_v)

        issue_gather(cp, 0, 0, 0)

        def pair_body(h, carry2, cp=cp):
            for jj in range(2):
                p = h * 2 + jj
                ob = 1 - jj

                def prefetch(h=h, jj=jj, ob=ob, cp=cp):
                    nh = h + jj
                    nj = 1 - jj

                    @pl.when(h * 2 + jj >= 1)
                    def _():
                        wait_scatter(ob, 0)
                        wait_scatter(ob, 1)

                    issue_gather(cp, nh, nj, ob)

                pl.when(p + 1 < NPAIR)(prefetch)
                wait_gather(cp, h, jj, jj)

                @plsc.parallel_loop(0, CS, 1, unroll=2)
                def add_row(r, jj=jj):
                    for k in range(KPR):
                        sl = pl.ds(k * LANES, LANES)
                        plsc.addupdate(rows[jj].at[r, sl], pe_v[r, sl])
                        plsc.addupdate(rows[jj].at[CS + r, sl], pe_v[r, sl])

                for half in range(2):
                    pltpu.async_copy(
                        rows[jj].at[pl.ds(half * CS, CS), :],
                        out_hbm.at[
                            wid * BPW + 2 * p + half, pl.ds(base, CS), :
                        ],
                        ssem[jj][half],
                    )
            return carry2

        lax.fori_loop(0, NPAIR // 2, pair_body, None)
        for jj in range(2):
            for half in range(2):
                wait_scatter(jj, half)
        return carry

    lax.fori_loop(0, NFULL, chunk_body, None)

    pltpu.sync_copy(pe_hbm.at[pl.ds(NFULL * CS, CS), :], pe_v)

    def tidx_list(q, jj):
        return idx_v.at[NFULL * GPC + q, pl.ds(jj * CS, CS)]

    def tail_quad(q, carry):
        for jj in range(4):
            seq = q * 4 + jj
            dst = rows0.at[pl.ds(0, CS), :]
            pltpu.async_copy(table_hbm.at[tidx_list(q, jj)], dst, gsem0)
            pltpu.make_async_copy(
                table_hbm.at[tidx_list(q, jj)], dst, gsem0
            ).wait()

            def add_row(r, cc):
                for k in range(KPR):
                    sl = pl.ds(k * LANES, LANES)
                    tail_v[r, sl] = rows0[r, sl] + pe_v[r, sl]
                return cc

            lax.fori_loop(0, TAIL, add_row, None)
            pltpu.sync_copy(
                tail_v, out_hbm.at[wid * BPW + seq, pl.ds(NFULL * CS, TAIL), :]
            )
        return carry

    lax.fori_loop(0, GPC, tail_quad, None)


def kernel(x, emb_table):
    pe = jnp.asarray(
        np.pad(_pe_matrix_np(D_MODEL, SEQ_LEN), ((0, SEQ_PAD - SEQ_LEN), (0, 0)))
    )
    xi = x.astype(jnp.int32)
    xc = (
        jnp.pad(xi, ((0, 0), (0, SEQ_PAD - SEQ_LEN)))
        .reshape(NW, BPW, NCHUNK, CS)
        .transpose(0, 2, 1, 3)
        .reshape(NW, NCHUNK * GPC, 4 * CS)
    )
    return _emb_pe_kernel(xc, emb_table, pe)
